# baseline (speedup 1.0000x reference)
import functools

import jax
import jax.numpy as jnp
from jax import lax
from jax.experimental import pallas as pl
from jax.experimental.pallas import tpu as pltpu
from jax.experimental.pallas import tpu_sc as plsc

_N = 10000
_E = 320000
_D = 128
_H = 128
_P = 50000
_U = 10000

_NC = 2
_NS = 16

_BN1 = 2000


def _stage1_body(x_ref, w_ref, b_ref, tab_ref, xiou_ref, fn_ref):
    xw = jnp.dot(x_ref[...], w_ref[...], preferred_element_type=jnp.float32)
    xw = xw + b_ref[...]
    xiou_ref[...] = xw[:, :768]
    fn_ref[...] = jax.nn.sigmoid(xw[:, 768:])
    for di in (0, 1):
        off = 384 * di
        i0 = jax.nn.sigmoid(xw[:, off:off + 128])
        o0 = jax.nn.sigmoid(xw[:, off + 128:off + 256])
        u0 = jnp.tanh(xw[:, off + 256:off + 384])
        c0 = i0 * u0
        tab_ref[2 * di] = c0
        tab_ref[2 * di + 1] = o0 * jnp.tanh(c0)


def _stage1(x, wcat, bcat):
    return pl.pallas_call(
        _stage1_body,
        grid=(_N // _BN1,),
        in_specs=[
            pl.BlockSpec((_BN1, _D), lambda i: (i, 0)),
            pl.BlockSpec((_D, 1024), lambda i: (0, 0)),
            pl.BlockSpec((1, 1024), lambda i: (0, 0)),
        ],
        out_specs=[
            pl.BlockSpec((4, _BN1, _D), lambda i: (0, i, 0)),
            pl.BlockSpec((_BN1, 768), lambda i: (i, 0)),
            pl.BlockSpec((_BN1, 256), lambda i: (i, 0)),
        ],
        out_shape=[
            jax.ShapeDtypeStruct((4, _N, _D), jnp.float32),
            jax.ShapeDtypeStruct((_N, 768), jnp.float32),
            jax.ShapeDtypeStruct((_N, 256), jnp.float32),
        ],
    )(x, wcat, bcat)


_ESUB = 128
_EPT = _E // _NS
_ESUBS = 158
_RPT = _N // _NS


_NHALF = _N // 2
_ACCR = _NHALF + 8
_RCH = 312
_ZCH = 328


def _edge_segsum(tabflat, gi4, si4, zrows):
    mesh = plsc.VectorSubcoreMesh(core_axis_name="c", subcore_axis_name="s",
                                  num_cores=_NC, num_subcores=_NS)

    @functools.partial(
        pl.kernel,
        out_type=jax.ShapeDtypeStruct((4 * _N, _D), jnp.float32),
        mesh=mesh,
        scratch_types=[
            pltpu.VMEM((_ESUBS, _ESUB), jnp.int32),
            pltpu.VMEM((_ESUBS, _ESUB), jnp.int32),
            pltpu.VMEM((2, _ESUB), jnp.int32),
            pltpu.VMEM((2, _ESUB, _D), jnp.float32),
            pltpu.VMEM_SHARED((_ACCR, _D), jnp.float32),
            pltpu.SemaphoreType.DMA,
            pltpu.SemaphoreType.DMA,
        ],
    )
    def body(tab_hbm, gi_hbm, si_hbm, z_hbm, out_hbm, gidx, sidx, sidx_t,
             rows, acc, sem0, sem1):
        cid = lax.axis_index("c")
        sid = lax.axis_index("s")
        sems = (sem0, sem1)
        row0 = pl.multiple_of(sid * _RCH, 8)

        pltpu.sync_copy(gi_hbm.at[cid * _NS + sid], gidx)
        pltpu.sync_copy(si_hbm.at[(1 - cid) * _NS + sid], sidx)

        def add_rows(delta):
            def addrow(j, c):
                for u in range(_ESUB // 16):
                    gidx[j, pl.ds(16 * u, 16)] = gidx[j, pl.ds(16 * u, 16)] + delta
                return c
            lax.fori_loop(0, _ESUBS, addrow, 0)

        add_rows(2 * cid * _N)

        for p in range(2):
            if p == 1:
                add_rows(_N)
            toff = (2 * cid + p) * _N
            for h in range(2):
                lo = h * _NHALF

                @pl.when(sid < _NS - 1)
                def _():
                    pltpu.sync_copy(z_hbm.at[pl.ds(0, _RCH)],
                                    acc.at[pl.ds(row0, _RCH)])

                @pl.when(sid == _NS - 1)
                def _():
                    pltpu.sync_copy(z_hbm,
                                    acc.at[pl.ds((_NS - 1) * _RCH, _ZCH)])

                plsc.subcore_barrier()

                def gather(j, b):
                    pltpu.async_copy(
                        tab_hbm.at[gidx.at[j]], rows.at[b], sems[b])

                def wait(b):
                    pltpu.make_async_copy(
                        tab_hbm.at[gidx.at[0]], rows.at[b], sems[b]).wait()

                def scatter(j, b):
                    for u in range(_ESUB // 16):
                        sl = pl.ds(16 * u, 16)
                        v = sidx[j, sl] - lo
                        ok = (v >= 0) & (v < _NHALF)
                        sidx_t[b, sl] = jnp.where(ok, v, _NHALF)
                    pltpu.sync_copy(rows.at[b], acc.at[sidx_t.at[b]], add=True)

                gather(0, 0)

                def step(i, c):
                    j0 = 2 * i
                    gather(j0 + 1, 1)
                    wait(0)
                    scatter(j0, 0)

                    @pl.when(i < _ESUBS // 2 - 1)
                    def _():
                        gather(j0 + 2, 0)

                    wait(1)
                    scatter(j0 + 1, 1)
                    return c

                lax.fori_loop(0, _ESUBS // 2, step, 0)
                plsc.subcore_barrier()
                orow = pl.multiple_of(toff + lo + row0, 8)

                @pl.when(sid < _NS - 1)
                def _():
                    pltpu.sync_copy(acc.at[pl.ds(row0, _RCH)],
                                    out_hbm.at[pl.ds(orow, _RCH)])

                @pl.when(sid == _NS - 1)
                def _():
                    tailr = _NHALF - (_NS - 1) * _RCH
                    pltpu.sync_copy(acc.at[pl.ds((_NS - 1) * _RCH, tailr)],
                                    out_hbm.at[pl.ds(orow, tailr)])

                plsc.subcore_barrier()

    return body(tabflat, gi4, si4, zrows)


_BN3 = 1000


def _stage3_body(x_ref, xiou_ref, fn_ref, s_ref, u_ref, w1_ref, tab_ref):
    hs = []
    for di in (0, 1):
        xiou = xiou_ref[:, 384 * di:384 * (di + 1)]
        aggh = s_ref[2 * di + 1]
        aggc = fn_ref[:, 128 * di:128 * (di + 1)] * s_ref[2 * di]
        iou = xiou + jnp.dot(aggh, u_ref[:, 384 * di:384 * (di + 1)],
                             preferred_element_type=jnp.float32)
        i = jax.nn.sigmoid(iou[:, :128])
        o = jax.nn.sigmoid(iou[:, 128:256])
        u = jnp.tanh(iou[:, 256:])
        c = i * u + aggc
        hs.append(o * jnp.tanh(c))
    h_f, h_b = hs
    x = x_ref[...]
    tab_ref[0] = jnp.dot(h_f, w1_ref[0:128], preferred_element_type=jnp.float32)
    tab_ref[1] = jnp.dot(h_b, w1_ref[128:256], preferred_element_type=jnp.float32)
    tab_ref[2] = jnp.dot(h_b, w1_ref[256:384], preferred_element_type=jnp.float32)
    tab_ref[3] = jnp.dot(x, w1_ref[384:512], preferred_element_type=jnp.float32)
    tab_ref[4] = jnp.dot(x, w1_ref[512:640], preferred_element_type=jnp.float32)


def _stage3(x, xiou, fn, s4, ucat, w1):
    return pl.pallas_call(
        _stage3_body,
        grid=(_N // _BN3,),
        in_specs=[
            pl.BlockSpec((_BN3, _D), lambda i: (i, 0)),
            pl.BlockSpec((_BN3, 768), lambda i: (i, 0)),
            pl.BlockSpec((_BN3, 256), lambda i: (i, 0)),
            pl.BlockSpec((4, _BN3, _D), lambda i: (0, i, 0)),
            pl.BlockSpec((_D, 768), lambda i: (0, 0)),
            pl.BlockSpec((640, 256), lambda i: (0, 0)),
        ],
        out_specs=pl.BlockSpec((5, _BN3, 256), lambda i: (0, i, 0)),
        out_shape=jax.ShapeDtypeStruct((5, _N, 256), jnp.float32),
    )(x, xiou, fn, s4, ucat, w1)


_PSUB = 80
_PPT = 1600
_PPAD = _NC * _NS * _PPT
_PSUBS = _PPT // _PSUB


def _pair_gather(tab5flat, root, start, end, p1g, p2g, unit_idx):
    mesh = plsc.VectorSubcoreMesh(core_axis_name="c", subcore_axis_name="s",
                                  num_cores=_NC, num_subcores=_NS)

    @functools.partial(
        pl.kernel,
        out_type=jax.ShapeDtypeStruct((_PPAD, 256), jnp.float32),
        mesh=mesh,
        scratch_types=[
            pltpu.VMEM((_PPT,), jnp.int32),
            pltpu.VMEM((_PPT,), jnp.int32),
            pltpu.VMEM((_PPT,), jnp.int32),
            pltpu.VMEM((_PPT,), jnp.int32),
            pltpu.VMEM((_PPT,), jnp.int32),
            pltpu.VMEM((_PPT,), jnp.int32),
            pltpu.VMEM((_PPT,), jnp.int32),
            pltpu.VMEM((5, _PSUB, 256), jnp.float32),
            pltpu.SemaphoreType.DMA,
            pltpu.SemaphoreType.DMA,
        ],
    )
    def body(tab_hbm, r_hbm, s_hbm, e_hbm, p1_hbm, p2_hbm, u_hbm, out_hbm,
             i0, i1, i2, i3, i4, q1b, q2b, gbuf, sem, semq):
        cid = lax.axis_index("c")
        sid = lax.axis_index("s")
        wid = sid * _NC + cid
        base = pl.multiple_of(wid * _PPT, 8)
        ibig = (i0, i1, i2, i3, i4)

        pltpu.sync_copy(r_hbm.at[pl.ds(base, _PPT)], i0)
        pltpu.sync_copy(s_hbm.at[pl.ds(base, _PPT)], i1)
        pltpu.sync_copy(e_hbm.at[pl.ds(base, _PPT)], i2)
        pltpu.sync_copy(p1_hbm.at[pl.ds(base, _PPT)], q1b)
        pltpu.sync_copy(p2_hbm.at[pl.ds(base, _PPT)], q2b)
        for k, (qb, dst) in enumerate(((q1b, i3), (q2b, i4))):
            for s in range(_PSUBS):
                soff = 80 * s
                pltpu.async_copy(
                    u_hbm.at[qb.at[pl.ds(soff, _PSUB)]],
                    dst.at[pl.ds(soff, _PSUB)], semq)
            for s in range(_PSUBS):
                pltpu.make_async_copy(
                    u_hbm.at[qb.at[pl.ds(0, _PSUB)]],
                    dst.at[pl.ds(0, _PSUB)], semq).wait()

        def addoff(g, c):
            sl = pl.ds(16 * g, 16)
            for t in range(1, 5):
                ibig[t][sl] = ibig[t][sl] + t * _N
            return c

        lax.fori_loop(0, _PPT // 16, addoff, 0)

        def sub(s, carry):
            soff = pl.multiple_of(s * _PSUB, 8)
            for t in range(5):
                pltpu.async_copy(
                    tab_hbm.at[ibig[t].at[pl.ds(soff, _PSUB)]], gbuf.at[t],
                    sem)
            for t in range(5):
                pltpu.make_async_copy(
                    tab_hbm.at[i0.at[pl.ds(0, _PSUB)]], gbuf.at[t],
                    sem).wait()

            def row(r, c2):
                for u in range(16):
                    sl = pl.ds(16 * u, 16)
                    gbuf[0, r, sl] = (gbuf[0, r, sl] + gbuf[1, r, sl]
                                      + gbuf[2, r, sl] + gbuf[3, r, sl]
                                      + gbuf[4, r, sl])
                return c2

            lax.fori_loop(0, _PSUB, row, 0)
            pltpu.sync_copy(gbuf.at[0],
                            out_hbm.at[pl.ds(pl.multiple_of(base + soff, 8),
                                             _PSUB)])
            return carry

        lax.fori_loop(0, _PSUBS, sub, 0)

    return body(tab5flat, root, start, end, p1g, p2g, unit_idx)


_BP = 1600


def _stage5_body(p_ref, b1_ref, wa_ref, ba_ref, wb_ref, bb_ref, out_ref):
    pre = jnp.tanh(p_ref[...] + b1_ref[...])
    la = jnp.dot(pre, wa_ref[...], preferred_element_type=jnp.float32) + ba_ref[...]
    lb = jnp.dot(pre, wb_ref[...], preferred_element_type=jnp.float32) + bb_ref[...]
    ea = jnp.exp(la - jnp.max(la, axis=1, keepdims=True))
    pa = ea / jnp.sum(ea, axis=1, keepdims=True)
    eb = jnp.exp(lb - jnp.max(lb, axis=1, keepdims=True))
    pb = eb / jnp.sum(eb, axis=1, keepdims=True)
    ma = jnp.max(pa, axis=1, keepdims=True)
    mb = jnp.max(pb, axis=1, keepdims=True)
    d = mb > ma
    sel = jnp.where(d, pb, pa)
    col = lax.broadcasted_iota(jnp.int32, sel.shape, 1)
    out_ref[...] = jnp.where(col == 3, d.astype(jnp.float32), sel)[:, :8]


def _stage5(pre_in, b1r, wa, ba, wb, bb):
    return pl.pallas_call(
        _stage5_body,
        grid=(_PPAD // _BP,),
        in_specs=[
            pl.BlockSpec((_BP, 256), lambda i: (i, 0)),
            pl.BlockSpec((1, 256), lambda i: (0, 0)),
            pl.BlockSpec((256, 128), lambda i: (0, 0)),
            pl.BlockSpec((1, 128), lambda i: (0, 0)),
            pl.BlockSpec((256, 128), lambda i: (0, 0)),
            pl.BlockSpec((1, 128), lambda i: (0, 0)),
        ],
        out_specs=pl.BlockSpec((_BP, 8), lambda i: (i, 0)),
        out_shape=jax.ShapeDtypeStruct((_PPAD, 8), jnp.float32),
    )(pre_in, b1r, wa, ba, wb, bb)


def kernel(node_embs, edge_index, root_idx, start_idx, end_idx, p1g, p2g,
           unit_idx, W_iou_f, U_iou_f, b_iou_f, W_f_f, b_f_f,
           W_iou_b, U_iou_b, b_iou_b, W_f_b, b_f_b,
           W1, b1, W2a, b2a, W2b, b2b):
    x = node_embs
    wcat = jnp.concatenate([W_iou_f, W_iou_b, W_f_f, W_f_b], axis=1)
    bcat = jnp.concatenate([b_iou_f, b_iou_b, b_f_f, b_f_b])[None, :]
    tab01, xiou, fn = _stage1(x, wcat, bcat)

    epad = _ESUBS * _ESUB - _EPT
    eib = edge_index.reshape(2, _NS, _EPT)
    gi4 = jnp.pad(eib, ((0, 0), (0, 0), (0, epad)),
                  constant_values=0).reshape(2 * _NS, _ESUBS, _ESUB)
    si4 = jnp.pad(eib, ((0, 0), (0, 0), (0, epad)),
                  constant_values=-1).reshape(2 * _NS, _ESUBS, _ESUB)
    zrows = jnp.zeros((_ZCH, _D), jnp.float32)
    s4 = _edge_segsum(tab01.reshape(4 * _N, _D), gi4, si4,
                      zrows).reshape(4, _N, _D)

    ucat = jnp.concatenate([U_iou_f, U_iou_b], axis=1)
    tab5 = _stage3(x, xiou, fn, s4, ucat, W1)

    pad = _PPAD - _P
    rootp = jnp.pad(root_idx, (0, pad))
    startp = jnp.pad(start_idx, (0, pad))
    endp = jnp.pad(end_idx, (0, pad))
    p1p = jnp.pad(p1g, (0, pad))
    p2p = jnp.pad(p2g, (0, pad))
    pre_in = _pair_gather(tab5.reshape(5 * _N, 256), rootp, startp, endp,
                          p1p, p2p, unit_idx)

    b1r = b1[None, :]
    wa = jnp.zeros((256, 128), jnp.float32).at[:, :3].set(W2a)
    wb = jnp.zeros((256, 128), jnp.float32).at[:, :3].set(W2b)
    ba = jnp.full((1, 128), -1e30, jnp.float32).at[0, :3].set(b2a)
    bb = jnp.full((1, 128), -1e30, jnp.float32).at[0, :3].set(b2b)
    out = _stage5(pre_in, b1r, wa, ba, wb, bb)

    pair_probs = out[:_P, :3]
    directions = out[:_P, 3].astype(jnp.int32)
    return pair_probs, directions


# speedup vs baseline: 1.3816x; 1.3816x over previous
import functools

import jax
import jax.numpy as jnp
from jax import lax
from jax.experimental import pallas as pl
from jax.experimental.pallas import tpu as pltpu
from jax.experimental.pallas import tpu_sc as plsc

_N = 10000
_E = 320000
_D = 128
_H = 128
_P = 50000
_U = 10000

_NC = 2
_NS = 16

_BN1 = 2000


def _stage1_body(x_ref, w_ref, b_ref, tab_ref, xiou_ref, fn_ref):
    xw = jnp.dot(x_ref[...], w_ref[...], preferred_element_type=jnp.float32)
    xw = xw + b_ref[...]
    xiou_ref[...] = xw[:, :768]
    fn_ref[...] = jax.nn.sigmoid(xw[:, 768:])
    for di in (0, 1):
        off = 384 * di
        i0 = jax.nn.sigmoid(xw[:, off:off + 128])
        o0 = jax.nn.sigmoid(xw[:, off + 128:off + 256])
        u0 = jnp.tanh(xw[:, off + 256:off + 384])
        c0 = i0 * u0
        tab_ref[2 * di] = c0
        tab_ref[2 * di + 1] = o0 * jnp.tanh(c0)


def _stage1(x, wcat, bcat):
    return pl.pallas_call(
        _stage1_body,
        grid=(_N // _BN1,),
        in_specs=[
            pl.BlockSpec((_BN1, _D), lambda i: (i, 0)),
            pl.BlockSpec((_D, 1024), lambda i: (0, 0)),
            pl.BlockSpec((1, 1024), lambda i: (0, 0)),
        ],
        out_specs=[
            pl.BlockSpec((4, _BN1, _D), lambda i: (0, i, 0)),
            pl.BlockSpec((_BN1, 768), lambda i: (i, 0)),
            pl.BlockSpec((_BN1, 256), lambda i: (i, 0)),
        ],
        out_shape=[
            jax.ShapeDtypeStruct((4, _N, _D), jnp.float32),
            jax.ShapeDtypeStruct((_N, 768), jnp.float32),
            jax.ShapeDtypeStruct((_N, 256), jnp.float32),
        ],
    )(x, wcat, bcat)


_ESUB = 80
_EPT = _E // _NS
_ESUBS = _EPT // _ESUB
_RPT = _N // _NS


_NHALF = _N // 2
_ACCR = _NHALF + 8
_RCH = 312
_ZCH = 328


def _edge_segsum(tabflat, ei3, zrows):
    mesh = plsc.VectorSubcoreMesh(core_axis_name="c", subcore_axis_name="s",
                                  num_cores=_NC, num_subcores=_NS)

    @functools.partial(
        pl.kernel,
        out_type=jax.ShapeDtypeStruct((4 * _N, _D), jnp.float32),
        mesh=mesh,
        scratch_types=[
            pltpu.VMEM((_ESUBS, _ESUB), jnp.int32),
            pltpu.VMEM((_ESUBS, _ESUB), jnp.int32),
            pltpu.VMEM((2, _ESUB), jnp.int32),
            pltpu.VMEM((2, _ESUB, _D), jnp.float32),
            pltpu.VMEM_SHARED((_ACCR, _D), jnp.float32),
            pltpu.SemaphoreType.DMA,
            pltpu.SemaphoreType.DMA,
        ],
    )
    def body(tab_hbm, ei_hbm, z_hbm, out_hbm, gidx, sidx, sidx_t,
             rows, acc, sem0, sem1):
        cid = lax.axis_index("c")
        sid = lax.axis_index("s")
        sems = (sem0, sem1)
        row0 = pl.multiple_of(sid * _RCH, 8)

        pltpu.sync_copy(ei_hbm.at[cid * _NS + sid], gidx)
        pltpu.sync_copy(ei_hbm.at[(1 - cid) * _NS + sid], sidx)

        def add_rows(delta):
            def addrow(j, c):
                for u in range(_ESUB // 16):
                    gidx[j, pl.ds(16 * u, 16)] = gidx[j, pl.ds(16 * u, 16)] + delta
                return c
            lax.fori_loop(0, _ESUBS, addrow, 0)

        add_rows(2 * cid * _N)

        for p in range(2):
            if p == 1:
                add_rows(_N)
            toff = (2 * cid + p) * _N
            for h in range(2):
                lo = h * _NHALF

                @pl.when(sid < _NS - 1)
                def _():
                    pltpu.sync_copy(z_hbm.at[pl.ds(0, _RCH)],
                                    acc.at[pl.ds(row0, _RCH)])

                @pl.when(sid == _NS - 1)
                def _():
                    pltpu.sync_copy(z_hbm,
                                    acc.at[pl.ds((_NS - 1) * _RCH, _ZCH)])

                plsc.subcore_barrier()

                def gather(j, b):
                    pltpu.async_copy(
                        tab_hbm.at[gidx.at[j]], rows.at[b], sems[b])

                def wait(b):
                    pltpu.make_async_copy(
                        tab_hbm.at[gidx.at[0]], rows.at[b], sems[b]).wait()

                def scatter(j, b):
                    for u in range(_ESUB // 16):
                        sl = pl.ds(16 * u, 16)
                        v = sidx[j, sl] - lo
                        ok = (v >= 0) & (v < _NHALF)
                        sidx_t[b, sl] = jnp.where(ok, v, _NHALF)
                    pltpu.sync_copy(rows.at[b], acc.at[sidx_t.at[b]], add=True)

                gather(0, 0)

                def step(i, c):
                    j0 = 2 * i
                    gather(j0 + 1, 1)
                    wait(0)
                    scatter(j0, 0)

                    @pl.when(i < _ESUBS // 2 - 1)
                    def _():
                        gather(j0 + 2, 0)

                    wait(1)
                    scatter(j0 + 1, 1)
                    return c

                lax.fori_loop(0, _ESUBS // 2, step, 0)
                plsc.subcore_barrier()
                orow = pl.multiple_of(toff + lo + row0, 8)

                @pl.when(sid < _NS - 1)
                def _():
                    pltpu.sync_copy(acc.at[pl.ds(row0, _RCH)],
                                    out_hbm.at[pl.ds(orow, _RCH)])

                @pl.when(sid == _NS - 1)
                def _():
                    tailr = _NHALF - (_NS - 1) * _RCH
                    pltpu.sync_copy(acc.at[pl.ds((_NS - 1) * _RCH, tailr)],
                                    out_hbm.at[pl.ds(orow, tailr)])

                plsc.subcore_barrier()

    return body(tabflat, ei3, zrows)


_BN3 = 1000


def _stage3_body(x_ref, xiou_ref, fn_ref, s_ref, u_ref, w1_ref, tab_ref):
    hs = []
    for di in (0, 1):
        xiou = xiou_ref[:, 384 * di:384 * (di + 1)]
        aggh = s_ref[2 * di + 1]
        aggc = fn_ref[:, 128 * di:128 * (di + 1)] * s_ref[2 * di]
        iou = xiou + jnp.dot(aggh, u_ref[:, 384 * di:384 * (di + 1)],
                             preferred_element_type=jnp.float32)
        i = jax.nn.sigmoid(iou[:, :128])
        o = jax.nn.sigmoid(iou[:, 128:256])
        u = jnp.tanh(iou[:, 256:])
        c = i * u + aggc
        hs.append(o * jnp.tanh(c))
    h_f, h_b = hs
    x = x_ref[...]
    tab_ref[0] = jnp.dot(h_f, w1_ref[0:128], preferred_element_type=jnp.float32)
    tab_ref[1] = jnp.dot(h_b, w1_ref[128:256], preferred_element_type=jnp.float32)
    tab_ref[2] = jnp.dot(h_b, w1_ref[256:384], preferred_element_type=jnp.float32)
    tab_ref[3] = jnp.dot(x, w1_ref[384:512], preferred_element_type=jnp.float32)
    tab_ref[4] = jnp.dot(x, w1_ref[512:640], preferred_element_type=jnp.float32)


def _stage3(x, xiou, fn, s4, ucat, w1):
    return pl.pallas_call(
        _stage3_body,
        grid=(_N // _BN3,),
        in_specs=[
            pl.BlockSpec((_BN3, _D), lambda i: (i, 0)),
            pl.BlockSpec((_BN3, 768), lambda i: (i, 0)),
            pl.BlockSpec((_BN3, 256), lambda i: (i, 0)),
            pl.BlockSpec((4, _BN3, _D), lambda i: (0, i, 0)),
            pl.BlockSpec((_D, 768), lambda i: (0, 0)),
            pl.BlockSpec((640, 256), lambda i: (0, 0)),
        ],
        out_specs=pl.BlockSpec((5, _BN3, 256), lambda i: (0, i, 0)),
        out_shape=jax.ShapeDtypeStruct((5, _N, 256), jnp.float32),
    )(x, xiou, fn, s4, ucat, w1)


_PSUB = 80
_PPT = 1600
_PPAD = _NC * _NS * _PPT
_PSUBS = _PPT // _PSUB


def _pair_gather(tab5flat, root, start, end, p1g, p2g, unit_idx):
    mesh = plsc.VectorSubcoreMesh(core_axis_name="c", subcore_axis_name="s",
                                  num_cores=_NC, num_subcores=_NS)

    @functools.partial(
        pl.kernel,
        out_type=jax.ShapeDtypeStruct((_PPAD, 256), jnp.float32),
        mesh=mesh,
        scratch_types=[
            pltpu.VMEM((_PPT,), jnp.int32),
            pltpu.VMEM((_PPT,), jnp.int32),
            pltpu.VMEM((_PPT,), jnp.int32),
            pltpu.VMEM((_PPT,), jnp.int32),
            pltpu.VMEM((_PPT,), jnp.int32),
            pltpu.VMEM((_PPT,), jnp.int32),
            pltpu.VMEM((_PPT,), jnp.int32),
            pltpu.VMEM((5, _PSUB, 256), jnp.float32),
            pltpu.SemaphoreType.DMA,
            pltpu.SemaphoreType.DMA,
        ],
    )
    def body(tab_hbm, r_hbm, s_hbm, e_hbm, p1_hbm, p2_hbm, u_hbm, out_hbm,
             i0, i1, i2, i3, i4, q1b, q2b, gbuf, sem, semq):
        cid = lax.axis_index("c")
        sid = lax.axis_index("s")
        wid = sid * _NC + cid
        base = pl.multiple_of(wid * _PPT, 8)
        ibig = (i0, i1, i2, i3, i4)

        pltpu.sync_copy(r_hbm.at[pl.ds(base, _PPT)], i0)
        pltpu.sync_copy(s_hbm.at[pl.ds(base, _PPT)], i1)
        pltpu.sync_copy(e_hbm.at[pl.ds(base, _PPT)], i2)
        pltpu.sync_copy(p1_hbm.at[pl.ds(base, _PPT)], q1b)
        pltpu.sync_copy(p2_hbm.at[pl.ds(base, _PPT)], q2b)
        for k, (qb, dst) in enumerate(((q1b, i3), (q2b, i4))):
            for s in range(_PSUBS):
                soff = 80 * s
                pltpu.async_copy(
                    u_hbm.at[qb.at[pl.ds(soff, _PSUB)]],
                    dst.at[pl.ds(soff, _PSUB)], semq)
            for s in range(_PSUBS):
                pltpu.make_async_copy(
                    u_hbm.at[qb.at[pl.ds(0, _PSUB)]],
                    dst.at[pl.ds(0, _PSUB)], semq).wait()

        def addoff(g, c):
            sl = pl.ds(16 * g, 16)
            for t in range(1, 5):
                ibig[t][sl] = ibig[t][sl] + t * _N
            return c

        lax.fori_loop(0, _PPT // 16, addoff, 0)

        def sub(s, carry):
            soff = pl.multiple_of(s * _PSUB, 8)
            for t in range(5):
                pltpu.async_copy(
                    tab_hbm.at[ibig[t].at[pl.ds(soff, _PSUB)]], gbuf.at[t],
                    sem)
            for t in range(5):
                pltpu.make_async_copy(
                    tab_hbm.at[i0.at[pl.ds(0, _PSUB)]], gbuf.at[t],
                    sem).wait()

            def row(r, c2):
                for u in range(16):
                    sl = pl.ds(16 * u, 16)
                    gbuf[0, r, sl] = (gbuf[0, r, sl] + gbuf[1, r, sl]
                                      + gbuf[2, r, sl] + gbuf[3, r, sl]
                                      + gbuf[4, r, sl])
                return c2

            lax.fori_loop(0, _PSUB, row, 0)
            pltpu.sync_copy(gbuf.at[0],
                            out_hbm.at[pl.ds(pl.multiple_of(base + soff, 8),
                                             _PSUB)])
            return carry

        lax.fori_loop(0, _PSUBS, sub, 0)

    return body(tab5flat, root, start, end, p1g, p2g, unit_idx)


_BP = 1600


def _stage5_body(p_ref, b1_ref, wa_ref, ba_ref, wb_ref, bb_ref, out_ref):
    pre = jnp.tanh(p_ref[...] + b1_ref[...])
    la = jnp.dot(pre, wa_ref[...], preferred_element_type=jnp.float32) + ba_ref[...]
    lb = jnp.dot(pre, wb_ref[...], preferred_element_type=jnp.float32) + bb_ref[...]
    ea = jnp.exp(la - jnp.max(la, axis=1, keepdims=True))
    pa = ea / jnp.sum(ea, axis=1, keepdims=True)
    eb = jnp.exp(lb - jnp.max(lb, axis=1, keepdims=True))
    pb = eb / jnp.sum(eb, axis=1, keepdims=True)
    ma = jnp.max(pa, axis=1, keepdims=True)
    mb = jnp.max(pb, axis=1, keepdims=True)
    d = mb > ma
    sel = jnp.where(d, pb, pa)
    col = lax.broadcasted_iota(jnp.int32, sel.shape, 1)
    out_ref[...] = jnp.where(col == 3, d.astype(jnp.float32), sel)[:, :8]


def _stage5(pre_in, b1r, wa, ba, wb, bb):
    return pl.pallas_call(
        _stage5_body,
        grid=(_PPAD // _BP,),
        in_specs=[
            pl.BlockSpec((_BP, 256), lambda i: (i, 0)),
            pl.BlockSpec((1, 256), lambda i: (0, 0)),
            pl.BlockSpec((256, 128), lambda i: (0, 0)),
            pl.BlockSpec((1, 128), lambda i: (0, 0)),
            pl.BlockSpec((256, 128), lambda i: (0, 0)),
            pl.BlockSpec((1, 128), lambda i: (0, 0)),
        ],
        out_specs=pl.BlockSpec((_BP, 8), lambda i: (i, 0)),
        out_shape=jax.ShapeDtypeStruct((_PPAD, 8), jnp.float32),
    )(pre_in, b1r, wa, ba, wb, bb)


def kernel(node_embs, edge_index, root_idx, start_idx, end_idx, p1g, p2g,
           unit_idx, W_iou_f, U_iou_f, b_iou_f, W_f_f, b_f_f,
           W_iou_b, U_iou_b, b_iou_b, W_f_b, b_f_b,
           W1, b1, W2a, b2a, W2b, b2b):
    x = node_embs
    wcat = jnp.concatenate([W_iou_f, W_iou_b, W_f_f, W_f_b], axis=1)
    bcat = jnp.concatenate([b_iou_f, b_iou_b, b_f_f, b_f_b])[None, :]
    tab01, xiou, fn = _stage1(x, wcat, bcat)

    ei3 = edge_index.reshape(2 * _NS, _ESUBS, _ESUB)
    zrows = jnp.zeros((_ZCH, _D), jnp.float32)
    s4 = _edge_segsum(tab01.reshape(4 * _N, _D), ei3,
                      zrows).reshape(4, _N, _D)

    ucat = jnp.concatenate([U_iou_f, U_iou_b], axis=1)
    tab5 = _stage3(x, xiou, fn, s4, ucat, W1)

    pad = _PPAD - _P
    rootp = jnp.pad(root_idx, (0, pad))
    startp = jnp.pad(start_idx, (0, pad))
    endp = jnp.pad(end_idx, (0, pad))
    p1p = jnp.pad(p1g, (0, pad))
    p2p = jnp.pad(p2g, (0, pad))
    pre_in = _pair_gather(tab5.reshape(5 * _N, 256), rootp, startp, endp,
                          p1p, p2p, unit_idx)

    b1r = b1[None, :]
    wa = jnp.zeros((256, 128), jnp.float32).at[:, :3].set(W2a)
    wb = jnp.zeros((256, 128), jnp.float32).at[:, :3].set(W2b)
    ba = jnp.full((1, 128), -1e30, jnp.float32).at[0, :3].set(b2a)
    bb = jnp.full((1, 128), -1e30, jnp.float32).at[0, :3].set(b2b)
    out = _stage5(pre_in, b1r, wa, ba, wb, bb)

    pair_probs = out[:_P, :3]
    directions = out[:_P, 3].astype(jnp.int32)
    return pair_probs, directions


# speedup vs baseline: 1.5169x; 1.0979x over previous
import functools

import jax
import jax.numpy as jnp
from jax import lax
from jax.experimental import pallas as pl
from jax.experimental.pallas import tpu as pltpu
from jax.experimental.pallas import tpu_sc as plsc

_N = 10000
_E = 320000
_D = 128
_H = 128
_P = 50000
_U = 10000

_NC = 2
_NS = 16

_BN1 = 2000


def _stage1_body(x_ref, w_ref, b_ref, tab_ref, xiou_ref, fn_ref):
    xw = jnp.dot(x_ref[...], w_ref[...], preferred_element_type=jnp.float32)
    xw = xw + b_ref[...]
    xiou_ref[...] = xw[:, :768]
    fn_ref[...] = jax.nn.sigmoid(xw[:, 768:])
    for di in (0, 1):
        off = 384 * di
        i0 = jax.nn.sigmoid(xw[:, off:off + 128])
        o0 = jax.nn.sigmoid(xw[:, off + 128:off + 256])
        u0 = jnp.tanh(xw[:, off + 256:off + 384])
        c0 = i0 * u0
        tab_ref[2 * di] = c0
        tab_ref[2 * di + 1] = o0 * jnp.tanh(c0)


def _stage1(x, wcat, bcat):
    return pl.pallas_call(
        _stage1_body,
        grid=(_N // _BN1,),
        in_specs=[
            pl.BlockSpec((_BN1, _D), lambda i: (i, 0)),
            pl.BlockSpec((_D, 1024), lambda i: (0, 0)),
            pl.BlockSpec((1, 1024), lambda i: (0, 0)),
        ],
        out_specs=[
            pl.BlockSpec((4, _BN1, _D), lambda i: (0, i, 0)),
            pl.BlockSpec((_BN1, 768), lambda i: (i, 0)),
            pl.BlockSpec((_BN1, 256), lambda i: (i, 0)),
        ],
        out_shape=[
            jax.ShapeDtypeStruct((4, _N, _D), jnp.float32),
            jax.ShapeDtypeStruct((_N, 768), jnp.float32),
            jax.ShapeDtypeStruct((_N, 256), jnp.float32),
        ],
    )(x, wcat, bcat)


_ESUB = 80
_EPT = _E // _NS
_ESUBS = _EPT // _ESUB
_RPT = _N // _NS


_NHALF = _N // 2
_ACCR = _NHALF + 8
_RCH = 312
_ZCH = 328


def _edge_segsum(tabflat, ei3, zrows):
    mesh = plsc.VectorSubcoreMesh(core_axis_name="c", subcore_axis_name="s",
                                  num_cores=_NC, num_subcores=_NS)

    @functools.partial(
        pl.kernel,
        out_type=jax.ShapeDtypeStruct((4 * _N, _D), jnp.float32),
        mesh=mesh,
        scratch_types=[
            pltpu.VMEM((_ESUBS, _ESUB), jnp.int32),
            pltpu.VMEM((_ESUBS, _ESUB), jnp.int32),
            pltpu.VMEM((2, _ESUB), jnp.int32),
            pltpu.VMEM((2, _ESUB, _D), jnp.float32),
            pltpu.VMEM_SHARED((_ACCR, _D), jnp.float32),
            pltpu.SemaphoreType.DMA,
            pltpu.SemaphoreType.DMA,
        ],
    )
    def body(tab_hbm, ei_hbm, z_hbm, out_hbm, gidx, sidx, sidx_t,
             rows, acc, sem0, sem1):
        cid = lax.axis_index("c")
        sid = lax.axis_index("s")
        sems = (sem0, sem1)
        row0 = pl.multiple_of(sid * _RCH, 8)

        pltpu.sync_copy(ei_hbm.at[cid * _NS + sid], gidx)
        pltpu.sync_copy(ei_hbm.at[(1 - cid) * _NS + sid], sidx)

        def add_rows(delta):
            def addrow(j, c):
                for u in range(_ESUB // 16):
                    gidx[j, pl.ds(16 * u, 16)] = gidx[j, pl.ds(16 * u, 16)] + delta
                return c
            lax.fori_loop(0, _ESUBS, addrow, 0)

        add_rows(2 * cid * _N)

        for p in range(2):
            if p == 1:
                add_rows(_N)
            toff = (2 * cid + p) * _N
            for h in range(2):
                lo = h * _NHALF

                @pl.when(sid < _NS - 1)
                def _():
                    pltpu.sync_copy(z_hbm.at[pl.ds(0, _RCH)],
                                    acc.at[pl.ds(row0, _RCH)])

                @pl.when(sid == _NS - 1)
                def _():
                    pltpu.sync_copy(z_hbm,
                                    acc.at[pl.ds((_NS - 1) * _RCH, _ZCH)])

                plsc.subcore_barrier()

                def gather(j, b):
                    pltpu.async_copy(
                        tab_hbm.at[gidx.at[j]], rows.at[b], sems[b])

                def wait(b):
                    pltpu.make_async_copy(
                        tab_hbm.at[gidx.at[0]], rows.at[b], sems[b]).wait()

                def scatter(j, b):
                    for u in range(_ESUB // 16):
                        sl = pl.ds(16 * u, 16)
                        v = sidx[j, sl] - lo
                        ok = (v >= 0) & (v < _NHALF)
                        sidx_t[b, sl] = jnp.where(ok, v, _NHALF)
                    pltpu.sync_copy(rows.at[b], acc.at[sidx_t.at[b]], add=True)

                gather(0, 0)

                def step(i, c):
                    j0 = 2 * i
                    gather(j0 + 1, 1)
                    wait(0)
                    scatter(j0, 0)

                    @pl.when(i < _ESUBS // 2 - 1)
                    def _():
                        gather(j0 + 2, 0)

                    wait(1)
                    scatter(j0 + 1, 1)
                    return c

                lax.fori_loop(0, _ESUBS // 2, step, 0)
                plsc.subcore_barrier()
                orow = pl.multiple_of(toff + lo + row0, 8)

                @pl.when(sid < _NS - 1)
                def _():
                    pltpu.sync_copy(acc.at[pl.ds(row0, _RCH)],
                                    out_hbm.at[pl.ds(orow, _RCH)])

                @pl.when(sid == _NS - 1)
                def _():
                    tailr = _NHALF - (_NS - 1) * _RCH
                    pltpu.sync_copy(acc.at[pl.ds((_NS - 1) * _RCH, tailr)],
                                    out_hbm.at[pl.ds(orow, tailr)])

                plsc.subcore_barrier()

    return body(tabflat, ei3, zrows)


_BN3 = 1000


def _stage3_body(x_ref, xiou_ref, fn_ref, s_ref, u_ref, w1_ref, tab_ref):
    hs = []
    for di in (0, 1):
        xiou = xiou_ref[:, 384 * di:384 * (di + 1)]
        aggh = s_ref[2 * di + 1]
        aggc = fn_ref[:, 128 * di:128 * (di + 1)] * s_ref[2 * di]
        iou = xiou + jnp.dot(aggh, u_ref[:, 384 * di:384 * (di + 1)],
                             preferred_element_type=jnp.float32)
        i = jax.nn.sigmoid(iou[:, :128])
        o = jax.nn.sigmoid(iou[:, 128:256])
        u = jnp.tanh(iou[:, 256:])
        c = i * u + aggc
        hs.append(o * jnp.tanh(c))
    h_f, h_b = hs
    x = x_ref[...]
    tab_ref[0] = jnp.dot(h_f, w1_ref[0:128], preferred_element_type=jnp.float32)
    tab_ref[1] = jnp.dot(h_b, w1_ref[128:256], preferred_element_type=jnp.float32)
    tab_ref[2] = jnp.dot(h_b, w1_ref[256:384], preferred_element_type=jnp.float32)
    tab_ref[3] = jnp.dot(x, w1_ref[384:512], preferred_element_type=jnp.float32)
    tab_ref[4] = jnp.dot(x, w1_ref[512:640], preferred_element_type=jnp.float32)


def _stage3(x, xiou, fn, s4, ucat, w1):
    return pl.pallas_call(
        _stage3_body,
        grid=(_N // _BN3,),
        in_specs=[
            pl.BlockSpec((_BN3, _D), lambda i: (i, 0)),
            pl.BlockSpec((_BN3, 768), lambda i: (i, 0)),
            pl.BlockSpec((_BN3, 256), lambda i: (i, 0)),
            pl.BlockSpec((4, _BN3, _D), lambda i: (0, i, 0)),
            pl.BlockSpec((_D, 768), lambda i: (0, 0)),
            pl.BlockSpec((640, 256), lambda i: (0, 0)),
        ],
        out_specs=pl.BlockSpec((5, _BN3, 256), lambda i: (0, i, 0)),
        out_shape=jax.ShapeDtypeStruct((5, _N, 256), jnp.float32),
    )(x, xiou, fn, s4, ucat, w1)


_PSUB = 40
_PPT = 1600
_PPAD = _NC * _NS * _PPT
_PSUBS = _PPT // _PSUB


def _pair_gather(tab5flat, root, start, end, p1g, p2g, unit_idx):
    mesh = plsc.VectorSubcoreMesh(core_axis_name="c", subcore_axis_name="s",
                                  num_cores=_NC, num_subcores=_NS)

    @functools.partial(
        pl.kernel,
        out_type=jax.ShapeDtypeStruct((_PPAD, 256), jnp.float32),
        mesh=mesh,
        scratch_types=[
            pltpu.VMEM((_PPT,), jnp.int32),
            pltpu.VMEM((_PPT,), jnp.int32),
            pltpu.VMEM((_PPT,), jnp.int32),
            pltpu.VMEM((_PPT,), jnp.int32),
            pltpu.VMEM((_PPT,), jnp.int32),
            pltpu.VMEM((_PPT,), jnp.int32),
            pltpu.VMEM((_PPT,), jnp.int32),
            pltpu.VMEM((2, 5, _PSUB, 256), jnp.float32),
            pltpu.SemaphoreType.DMA,
            pltpu.SemaphoreType.DMA,
            pltpu.SemaphoreType.DMA,
        ],
    )
    def body(tab_hbm, r_hbm, s_hbm, e_hbm, p1_hbm, p2_hbm, u_hbm, out_hbm,
             i0, i1, i2, i3, i4, q1b, q2b, gbuf, sem0, sem1, semq):
        cid = lax.axis_index("c")
        sid = lax.axis_index("s")
        wid = sid * _NC + cid
        base = pl.multiple_of(wid * _PPT, 8)
        ibig = (i0, i1, i2, i3, i4)
        gsems = (sem0, sem1)

        pltpu.sync_copy(r_hbm.at[pl.ds(base, _PPT)], i0)
        pltpu.sync_copy(s_hbm.at[pl.ds(base, _PPT)], i1)
        pltpu.sync_copy(e_hbm.at[pl.ds(base, _PPT)], i2)
        pltpu.sync_copy(p1_hbm.at[pl.ds(base, _PPT)], q1b)
        pltpu.sync_copy(p2_hbm.at[pl.ds(base, _PPT)], q2b)
        for k, (qb, dst) in enumerate(((q1b, i3), (q2b, i4))):
            for s in range(_PPT // 80):
                soff = 80 * s
                pltpu.async_copy(
                    u_hbm.at[qb.at[pl.ds(soff, 80)]],
                    dst.at[pl.ds(soff, 80)], semq)
            for s in range(_PPT // 80):
                pltpu.make_async_copy(
                    u_hbm.at[qb.at[pl.ds(0, 80)]],
                    dst.at[pl.ds(0, 80)], semq).wait()

        def addoff(g, c):
            sl = pl.ds(16 * g, 16)
            for t in range(1, 5):
                ibig[t][sl] = ibig[t][sl] + t * _N
            return c

        lax.fori_loop(0, _PPT // 16, addoff, 0)

        def fire(s, d):
            soff = pl.multiple_of(s * _PSUB, 8)
            for t in range(5):
                pltpu.async_copy(
                    tab_hbm.at[ibig[t].at[pl.ds(soff, _PSUB)]],
                    gbuf.at[d, t], gsems[d])

        def drain(d):
            for t in range(5):
                pltpu.make_async_copy(
                    tab_hbm.at[i0.at[pl.ds(0, _PSUB)]], gbuf.at[d, t],
                    gsems[d]).wait()

        def sumout(s, d):
            def row(r, c2):
                for u in range(16):
                    sl = pl.ds(16 * u, 16)
                    gbuf[d, 0, r, sl] = (
                        gbuf[d, 0, r, sl] + gbuf[d, 1, r, sl]
                        + gbuf[d, 2, r, sl] + gbuf[d, 3, r, sl]
                        + gbuf[d, 4, r, sl])
                return c2

            lax.fori_loop(0, _PSUB, row, 0)
            off = pl.multiple_of(base + s * _PSUB, 8)
            pltpu.sync_copy(gbuf.at[d, 0], out_hbm.at[pl.ds(off, _PSUB)])

        fire(0, 0)

        def pairstep(i, c):
            s0 = 2 * i
            fire(s0 + 1, 1)
            drain(0)
            sumout(s0, 0)

            @pl.when(i < _PSUBS // 2 - 1)
            def _():
                fire(s0 + 2, 0)

            drain(1)
            sumout(s0 + 1, 1)
            return c

        lax.fori_loop(0, _PSUBS // 2, pairstep, 0)

    return body(tab5flat, root, start, end, p1g, p2g, unit_idx)


_BP = 1600


def _stage5_body(p_ref, b1_ref, wa_ref, ba_ref, wb_ref, bb_ref, out_ref):
    pre = jnp.tanh(p_ref[...] + b1_ref[...])
    la = jnp.dot(pre, wa_ref[...], preferred_element_type=jnp.float32) + ba_ref[...]
    lb = jnp.dot(pre, wb_ref[...], preferred_element_type=jnp.float32) + bb_ref[...]
    ea = jnp.exp(la - jnp.max(la, axis=1, keepdims=True))
    pa = ea / jnp.sum(ea, axis=1, keepdims=True)
    eb = jnp.exp(lb - jnp.max(lb, axis=1, keepdims=True))
    pb = eb / jnp.sum(eb, axis=1, keepdims=True)
    ma = jnp.max(pa, axis=1, keepdims=True)
    mb = jnp.max(pb, axis=1, keepdims=True)
    d = mb > ma
    sel = jnp.where(d, pb, pa)
    col = lax.broadcasted_iota(jnp.int32, sel.shape, 1)
    out_ref[...] = jnp.where(col == 3, d.astype(jnp.float32), sel)[:, :8]


def _stage5(pre_in, b1r, wa, ba, wb, bb):
    return pl.pallas_call(
        _stage5_body,
        grid=(_PPAD // _BP,),
        in_specs=[
            pl.BlockSpec((_BP, 256), lambda i: (i, 0)),
            pl.BlockSpec((1, 256), lambda i: (0, 0)),
            pl.BlockSpec((256, 128), lambda i: (0, 0)),
            pl.BlockSpec((1, 128), lambda i: (0, 0)),
            pl.BlockSpec((256, 128), lambda i: (0, 0)),
            pl.BlockSpec((1, 128), lambda i: (0, 0)),
        ],
        out_specs=pl.BlockSpec((_BP, 8), lambda i: (i, 0)),
        out_shape=jax.ShapeDtypeStruct((_PPAD, 8), jnp.float32),
    )(pre_in, b1r, wa, ba, wb, bb)


def kernel(node_embs, edge_index, root_idx, start_idx, end_idx, p1g, p2g,
           unit_idx, W_iou_f, U_iou_f, b_iou_f, W_f_f, b_f_f,
           W_iou_b, U_iou_b, b_iou_b, W_f_b, b_f_b,
           W1, b1, W2a, b2a, W2b, b2b):
    x = node_embs
    wcat = jnp.concatenate([W_iou_f, W_iou_b, W_f_f, W_f_b], axis=1)
    bcat = jnp.concatenate([b_iou_f, b_iou_b, b_f_f, b_f_b])[None, :]
    tab01, xiou, fn = _stage1(x, wcat, bcat)

    ei3 = edge_index.reshape(2 * _NS, _ESUBS, _ESUB)
    zrows = jnp.zeros((_ZCH, _D), jnp.float32)
    s4 = _edge_segsum(tab01.reshape(4 * _N, _D), ei3,
                      zrows).reshape(4, _N, _D)

    ucat = jnp.concatenate([U_iou_f, U_iou_b], axis=1)
    tab5 = _stage3(x, xiou, fn, s4, ucat, W1)

    pad = _PPAD - _P
    rootp = jnp.pad(root_idx, (0, pad))
    startp = jnp.pad(start_idx, (0, pad))
    endp = jnp.pad(end_idx, (0, pad))
    p1p = jnp.pad(p1g, (0, pad))
    p2p = jnp.pad(p2g, (0, pad))
    pre_in = _pair_gather(tab5.reshape(5 * _N, 256), rootp, startp, endp,
                          p1p, p2p, unit_idx)

    b1r = b1[None, :]
    wa = jnp.zeros((256, 128), jnp.float32).at[:, :3].set(W2a)
    wb = jnp.zeros((256, 128), jnp.float32).at[:, :3].set(W2b)
    ba = jnp.full((1, 128), -1e30, jnp.float32).at[0, :3].set(b2a)
    bb = jnp.full((1, 128), -1e30, jnp.float32).at[0, :3].set(b2b)
    out = _stage5(pre_in, b1r, wa, ba, wb, bb)

    pair_probs = out[:_P, :3]
    directions = out[:_P, 3].astype(jnp.int32)
    return pair_probs, directions
